# baseline (speedup 1.0000x reference)
import functools

import jax
import jax.numpy as jnp
from jax.experimental import pallas as pl
from jax.experimental.pallas import tpu as pltpu

_BLK = 256


def _smoothness_body(a0_hbm, a1_hbm, z_ref, out_ref, buf, sem, *, inv_n, npairs):
    zfull = z_ref[...]

    def start_pair(j, s0):
        pltpu.make_async_copy(
            a0_hbm.at[pl.ds(2 * j * _BLK, _BLK), :], buf.at[s0], sem.at[s0]
        ).start()
        pltpu.make_async_copy(
            a1_hbm.at[pl.ds((2 * j + 1) * _BLK, _BLK), :],
            buf.at[s0 + 1],
            sem.at[s0 + 1],
        ).start()

    start_pair(0, 0)
    start_pair(1, 2)

    def chunk(slot, row0, acc):
        a = buf[slot]
        zi = z_ref[pl.ds(row0 * _BLK, _BLK), :]
        y = jnp.dot(a, zfull, preferred_element_type=jnp.float32)
        d = jnp.sum(a, axis=1)
        s = jnp.sum(zi * zi, axis=1)
        return acc + jnp.sum(d * s) - jnp.sum(zi * y)

    def step(j, acc):
        s0 = 2 * jax.lax.rem(j, 2)
        pltpu.make_async_copy(
            a0_hbm.at[pl.ds(2 * j * _BLK, _BLK), :], buf.at[s0], sem.at[s0]
        ).wait()
        acc = chunk(s0, 2 * j, acc)
        pltpu.make_async_copy(
            a1_hbm.at[pl.ds((2 * j + 1) * _BLK, _BLK), :],
            buf.at[s0 + 1],
            sem.at[s0 + 1],
        ).wait()
        acc = chunk(s0 + 1, 2 * j + 1, acc)

        @pl.when(j + 2 < npairs)
        def _():
            start_pair(j + 2, s0)

        return acc

    acc = jax.lax.fori_loop(0, npairs, step, jnp.float32(0.0))
    out_ref[...] = jnp.reshape(acc * inv_n, (1, 1))


@jax.jit
def kernel(z, coords, precomputed_adj):
    del coords
    n, dim = z.shape
    npairs = n // (2 * _BLK)
    out = pl.pallas_call(
        functools.partial(_smoothness_body, inv_n=1.0 / n, npairs=npairs),
        in_specs=[
            pl.BlockSpec(memory_space=pltpu.MemorySpace.HBM),
            pl.BlockSpec(memory_space=pltpu.MemorySpace.HBM),
            pl.BlockSpec(memory_space=pltpu.MemorySpace.VMEM),
        ],
        out_specs=pl.BlockSpec(memory_space=pltpu.MemorySpace.VMEM),
        out_shape=jax.ShapeDtypeStruct((1, 1), jnp.float32),
        scratch_shapes=[
            pltpu.VMEM((4, _BLK, 4096), jnp.float32),
            pltpu.SemaphoreType.DMA((4,)),
        ],
    )(precomputed_adj, precomputed_adj, z)
    return out[0, 0]
